# baseline (speedup 1.0000x reference)
import functools

import jax
import jax.numpy as jnp
import numpy as np
from jax import lax
from jax.experimental import pallas as pl
from jax.experimental.pallas import tpu as pltpu
from jax.experimental.pallas import tpu_sc as plsc

N = 10000
E = 320000
D = 128
DH = D // 2
NC = 2
NS = 16
NW = NC * NS
EPW = E // NW
CHUNK = 80
GC = EPW // CHUNK
PACK = 1 << 14

_PERM = np.concatenate(
    [np.concatenate([32 * j + 2 * np.arange(16),
                     32 * j + 2 * np.arange(16) + 1]) for j in range(4)])


def _sc_spmm(xi32, packed_idx, vals):
  mesh = plsc.VectorSubcoreMesh(
      core_axis_name="c", subcore_axis_name="s", num_cores=NC,
      num_subcores=NS)

  @functools.partial(
      pl.kernel,
      out_type=jax.ShapeDtypeStruct((NC, N, D), jnp.float32),
      mesh=mesh,
      scratch_types=[
          pltpu.VMEM((EPW,), jnp.int32),
          pltpu.VMEM((CHUNK, DH), jnp.int32),
          pltpu.VMEM((CHUNK, DH), jnp.int32),
          pltpu.VMEM((CHUNK, DH), jnp.int32),
          pltpu.VMEM((CHUNK, D), jnp.float32),
          pltpu.VMEM((CHUNK, D), jnp.float32),
          pltpu.VMEM((CHUNK,), jnp.float32),
          pltpu.VMEM((CHUNK,), jnp.float32),
          pltpu.VMEM((CHUNK,), jnp.float32),
          pltpu.VMEM_SHARED((N, D), jnp.float32),
          pltpu.SemaphoreType.DMA,
          pltpu.SemaphoreType.DMA,
          pltpu.SemaphoreType.DMA,
      ],
      compiler_params=pltpu.CompilerParams(use_tc_tiling_on_sc=False),
  )
  def k(xi_hbm, pidx_hbm, vals_hbm, out_hbm,
        pidx_v, gb0, gb1, gb2, sb0, sb1, vb0, vb1, vb2, acc_sh,
        gsem, ssem, vsem):
    c = lax.axis_index("c")
    s = lax.axis_index("s")
    wid = s * NC + c
    gbufs = (gb0, gb1, gb2)
    sbufs = (sb0, sb1)
    vbufs = (vb0, vb1, vb2)

    nblk = N // CHUNK

    def _each_tile_block(fn):
      for kk in range((nblk + NS - 1) // NS):
        b = s + kk * NS

        @pl.when(b < nblk)
        def _(b=b):
          fn(b * CHUNK)

    zeros16 = jnp.zeros((16,), jnp.float32)

    @pl.loop(0, CHUNK)
    def _(e):
      for j in range(D // 16):
        sb0[e, pl.ds(j * 16, 16)] = zeros16

    _each_tile_block(
        lambda r0: pltpu.async_copy(
            sb0, acc_sh.at[pl.ds(r0, CHUNK), :], ssem))
    _each_tile_block(
        lambda r0: pltpu.make_async_copy(
            sb0, acc_sh.at[pl.ds(r0, CHUNK), :], ssem).wait())

    base = wid * EPW
    pltpu.sync_copy(pidx_hbm.at[pl.ds(base, EPW)], pidx_v)
    plsc.subcore_barrier()

    def _val_start(g, vb):
      pltpu.async_copy(vals_hbm.at[pl.ds(base + g * CHUNK, CHUNK)],
                       vb, vsem)

    def _val_wait(vb):
      pltpu.make_async_copy(vals_hbm.at[pl.ds(base, CHUNK)],
                            vb, vsem).wait()

    def _gather_start(g, gb):
      for t in range(CHUNK // 16):
        pk = pidx_v[pl.ds(g * CHUNK + t * 16, 16)]
        idx = jnp.bitwise_and(pk, PACK - 1)
        pltpu.async_copy(xi_hbm.at[idx], gb.at[pl.ds(t * 16, 16), :],
                         gsem)

    def _gather_wait(gb):
      for t in range(CHUNK // 16):
        pk = pidx_v[pl.ds(t * 16, 16)]
        idx = jnp.bitwise_and(pk, PACK - 1)
        pltpu.make_async_copy(xi_hbm.at[idx],
                              gb.at[pl.ds(t * 16, 16), :], gsem).wait()

    def _scatter_start(g, sb):
      for t in range(CHUNK // 16):
        pk = pidx_v[pl.ds(g * CHUNK + t * 16, 16)]
        idx = lax.shift_right_logical(pk, 14)
        pltpu.async_copy(sb.at[pl.ds(t * 16, 16), :],
                         acc_sh.at[idx], ssem, add=True)

    def _scatter_wait(sb):
      for t in range(CHUNK // 16):
        pk = pidx_v[pl.ds(t * 16, 16)]
        idx = lax.shift_right_logical(pk, 14)
        pltpu.make_async_copy(sb.at[pl.ds(t * 16, 16), :],
                              acc_sh.at[idx], ssem).wait()

    def _scale(g, gb, vb, sb):
      del g
      for t in range(CHUNK // 16):
        vv = vb[pl.ds(t * 16, 16)]
        for l in range(16):
          e = t * 16 + l
          v = vv[l]
          for j in range(DH // 16):
            pk = gb[e, pl.ds(j * 16, 16)]
            a = lax.bitcast_convert_type(
                jnp.left_shift(pk, 16), jnp.float32)
            b = lax.bitcast_convert_type(
                jnp.bitwise_and(pk, jnp.int32(-65536)), jnp.float32)
            sb[e, pl.ds(j * 32, 16)] = a * v
            sb[e, pl.ds(j * 32 + 16, 16)] = b * v

    def _step(g, gi, si, vi, wait_scatter, issue_ahead):
      _gather_wait(gbufs[gi])
      if wait_scatter:
        _scatter_wait(sbufs[si])
      if issue_ahead:
        _gather_start(g + 2, gbufs[(gi + 2) % 3])
        _val_start(g + 2, vbufs[(vi + 2) % 3])
      _val_wait(vbufs[vi])
      _scale(g, gbufs[gi], vbufs[vi], sbufs[si])
      _scatter_start(g, sbufs[si])

    _val_start(0, vb0)
    _val_start(1, vb1)
    _gather_start(0, gb0)
    _gather_start(1, gb1)
    _step(0, 0, 0, 0, False, True)
    _step(1, 1, 1, 1, False, True)

    @pl.loop(2, 122, step=6)
    def _(g0):
      for h in range(6):
        g = g0 + h
        _step(g, (2 + h) % 3, h % 2, (2 + h) % 3, True, True)

    for g in range(122, GC):
      _step(g, g % 3, g % 2, g % 3, True, g + 2 < GC)

    _scatter_wait(sbufs[(GC - 2) % 2])
    _scatter_wait(sbufs[(GC - 1) % 2])
    plsc.subcore_barrier()

    _each_tile_block(
        lambda r0: pltpu.async_copy(
            acc_sh.at[pl.ds(r0, CHUNK), :],
            out_hbm.at[c, pl.ds(r0, CHUNK), :], gsem))
    _each_tile_block(
        lambda r0: pltpu.make_async_copy(
            acc_sh.at[pl.ds(r0, CHUNK), :],
            out_hbm.at[c, pl.ds(r0, CHUNK), :], gsem).wait())

  return k(xi32, packed_idx, vals)


MB = 256


def _tc_body(b1_ref, p_ref, w_ref, relu_ref, z1_ref):
  psum = p_ref[0] + p_ref[1]
  t = jnp.dot(b1_ref[...], psum, preferred_element_type=jnp.float32)
  z1 = jnp.dot(t, w_ref[...], preferred_element_type=jnp.float32)
  z1_ref[...] = z1
  relu_ref[...] = jnp.maximum(z1, 0.0)


def _tc_matmuls(B_1, partials, W_perm):
  nb = B_1.shape[0]
  grid = nb // MB
  return pl.pallas_call(
      _tc_body,
      grid=(grid,),
      in_specs=[
          pl.BlockSpec((MB, N), lambda i: (i, 0)),
          pl.BlockSpec((NC, N, D), lambda i: (0, 0, 0)),
          pl.BlockSpec((D, D), lambda i: (0, 0)),
      ],
      out_specs=[
          pl.BlockSpec((MB, D), lambda i: (i, 0)),
          pl.BlockSpec((MB, D), lambda i: (i, 0)),
      ],
      out_shape=[
          jax.ShapeDtypeStruct((nb, D), jnp.float32),
          jax.ShapeDtypeStruct((nb, D), jnp.float32),
      ],
      compiler_params=pltpu.CompilerParams(
          dimension_semantics=("arbitrary",)),
  )(B_1, partials, W_perm)


def kernel(x, support_indices, support_values, B_1, W):
  xi32 = lax.bitcast_convert_type(
      x.astype(jnp.bfloat16).reshape(N, DH, 2), jnp.int32)
  packed = support_indices[0] * PACK + support_indices[1]
  W_perm = W[jnp.asarray(_PERM), :]
  partials = _sc_spmm(xi32, packed, support_values)
  relu_out, z1 = _tc_matmuls(B_1, partials, W_perm)
  return (relu_out, z1)
